# baseline (device time: 11597 ns/iter reference)
import jax
import jax.numpy as jnp
from jax import lax
from jax.experimental import pallas as pl
from jax.experimental.pallas import tpu as pltpu

N_DEV = 4


def kernel(x):
    m, n = x.shape
    sa = m // 4
    sb = m - sa

    def body(x_ref, out_ref, top_halo, bot_halo, send_sems, recv_sems):
        my = lax.axis_index("i")
        left = (my - 1) % N_DEV
        right = (my + 1) % N_DEV

        barrier_sem = pltpu.get_barrier_semaphore()
        for nbr in (left, right):
            pl.semaphore_signal(
                barrier_sem, inc=1,
                device_id=(nbr,), device_id_type=pl.DeviceIdType.MESH,
            )

        def rolled_stencil(lo, rows):
            xv = x_ref[pl.ds(lo, rows), :]
            out_ref[pl.ds(lo, rows), :] = (
                0.25 * pltpu.roll(xv, 1, 0)
                + 0.5 * xv
                + 0.25 * pltpu.roll(xv, rows - 1, 0)
            )

        def exact_row(r):
            out_ref[pl.ds(r, 1), :] = (
                0.25 * x_ref[pl.ds(r - 1, 1), :]
                + 0.5 * x_ref[pl.ds(r, 1), :]
                + 0.25 * x_ref[pl.ds(r + 1, 1), :]
            )

        rolled_stencil(0, sa)

        pl.semaphore_wait(barrier_sem, 2)
        up = pltpu.make_async_remote_copy(
            src_ref=x_ref.at[pl.ds(m - 1, 1)],
            dst_ref=top_halo,
            send_sem=send_sems.at[0],
            recv_sem=recv_sems.at[0],
            device_id=(right,),
            device_id_type=pl.DeviceIdType.MESH,
        )
        down = pltpu.make_async_remote_copy(
            src_ref=x_ref.at[pl.ds(0, 1)],
            dst_ref=bot_halo,
            send_sem=send_sems.at[1],
            recv_sem=recv_sems.at[1],
            device_id=(left,),
            device_id_type=pl.DeviceIdType.MESH,
        )
        up.start()
        down.start()

        rolled_stencil(sa, sb)

        exact_row(sa - 1)
        exact_row(sa)
        out_ref[pl.ds(0, 1), :] = x_ref[pl.ds(0, 1), :]
        out_ref[pl.ds(m - 1, 1), :] = x_ref[pl.ds(m - 1, 1), :]

        up.wait_recv()
        down.wait_recv()

        @pl.when(my > 0)
        def _():
            out_ref[pl.ds(0, 1), :] = (
                0.25 * top_halo[...]
                + 0.5 * x_ref[pl.ds(0, 1), :]
                + 0.25 * x_ref[pl.ds(1, 1), :]
            )

        @pl.when(my < N_DEV - 1)
        def _():
            out_ref[pl.ds(m - 1, 1), :] = (
                0.25 * x_ref[pl.ds(m - 2, 1), :]
                + 0.5 * x_ref[pl.ds(m - 1, 1), :]
                + 0.25 * bot_halo[...]
            )

        up.wait_send()
        down.wait_send()

    return pl.pallas_call(
        body,
        out_shape=jax.ShapeDtypeStruct((m, n), x.dtype),
        in_specs=[pl.BlockSpec(memory_space=pltpu.VMEM)],
        out_specs=pl.BlockSpec(memory_space=pltpu.VMEM),
        scratch_shapes=[
            pltpu.VMEM((1, n), x.dtype),
            pltpu.VMEM((1, n), x.dtype),
            pltpu.SemaphoreType.DMA((2,)),
            pltpu.SemaphoreType.DMA((2,)),
        ],
        compiler_params=pltpu.CompilerParams(collective_id=0),
    )(x)
